# baseline (speedup 1.0000x reference)
import functools

import numpy as np
import jax
import jax.numpy as jnp
from jax import lax
from jax.experimental import pallas as pl
from jax.experimental.pallas import tpu as pltpu
from jax.experimental.pallas import tpu_sc as plsc

_H = 768
_F = 1536
_E = 8
_T = 2048
_TILE = 256
_M = 6144
_NT = _M // _TILE
_F_BLK = 768
_NW = 32
_ROWS_W = _M // _NW
_GCHUNK = 32
_TOK_W = _T // _NW
_CCHUNK = 16

_TRI = np.tril(np.ones((512, 512), np.float32), -1)
_LI = np.tril(np.ones((8, 8), np.float32), 0)

_SC_PARAMS = pltpu.CompilerParams(
    needs_layout_passes=False,
    use_tc_tiling_on_sc=False,
)


def _router_body(x_ref, gw_ref, tri_ref, li_ref, pos1_ref, pos2_ref,
                 w1_ref, w2_ref, se_ref):
    x = x_ref[...]
    logits = lax.dot_general(x, gw_ref[...], (((1,), (1,)), ((), ())),
                             preferred_element_type=jnp.float32)
    p = jax.nn.softmax(logits, axis=-1)
    lane = lax.broadcasted_iota(jnp.int32, p.shape, 1)
    i1 = jnp.argmax(p, axis=-1)
    m1h = lane == i1[:, None]
    m1 = jnp.max(p, axis=-1, keepdims=True)
    p2 = jnp.where(m1h, -jnp.inf, p)
    i2 = jnp.argmax(p2, axis=-1)
    m2h = lane == i2[:, None]
    m2 = jnp.max(p2, axis=-1, keepdims=True)
    denom = m1 + m2 + 1e-8
    w1_ref[...] = m1 / denom
    w2_ref[...] = m2 / denom

    oh1 = m1h.astype(jnp.float32)
    oh2 = m2h.astype(jnp.float32)
    tri = tri_ref[...]
    carry = jnp.zeros((1, _E), jnp.float32)
    ranks = []
    for blk in range(8):
        oh = oh1 if blk < 4 else oh2
        ohb = oh[(blk % 4) * 512:(blk % 4) * 512 + 512, :]
        part = lax.dot_general(tri, ohb.astype(jnp.bfloat16),
                               (((1,), (0,)), ((), ())),
                               preferred_element_type=jnp.float32)
        ranks.append(part + carry)
        carry = carry + jnp.sum(ohb, axis=0, keepdims=True)
    rank1 = jnp.concatenate(ranks[0:4], axis=0)
    rank2 = jnp.concatenate(ranks[4:8], axis=0)

    ones_col = jnp.ones((_T, 1), jnp.float32)
    c_col = lax.dot_general(oh1 + oh2, ones_col, (((0,), (0,)), ((), ())),
                            preferred_element_type=jnp.float32)
    pc_col = jnp.ceil(c_col / _TILE) * _TILE
    pend_col = lax.dot_general(li_ref[...], pc_col, (((1,), (0,)), ((), ())),
                               preferred_element_type=jnp.float32)
    pstart_col = pend_col - pc_col

    base1 = lax.dot_general(oh1, pstart_col, (((1,), (0,)), ((), ())),
                            preferred_element_type=jnp.float32)
    base2 = lax.dot_general(oh2, pstart_col, (((1,), (0,)), ((), ())),
                            preferred_element_type=jnp.float32)
    r1 = jnp.sum(oh1 * rank1, axis=1, keepdims=True)
    r2 = jnp.sum(oh2 * rank2, axis=1, keepdims=True)
    pos1_ref[...] = (base1 + r1).astype(jnp.int32)
    pos2_ref[...] = (base2 + r2).astype(jnp.int32)

    m_iota = lax.broadcasted_iota(jnp.int32, (_E, 64), 1).astype(
        jnp.float32)
    hit = (pend_col <= m_iota * _TILE).astype(jnp.float32)
    te = lax.dot_general(jnp.ones((1, _E), jnp.float32), hit,
                         (((1,), (0,)), ((), ())),
                         preferred_element_type=jnp.float32)
    te = jnp.minimum(te, float(_E - 1))
    mrow = lax.broadcasted_iota(jnp.int32, (1, 64), 1)
    tei = te.astype(jnp.int32)
    e7 = (lax.broadcasted_iota(jnp.int32, (1, _E), 1) == _E - 1
          ).astype(jnp.float32)
    nr = lax.dot_general(e7, pend_col, (((1,), (0,)), ((), ())),
                         preferred_element_type=jnp.float32) / _TILE
    tei = jnp.where(mrow == _NT, nr.astype(jnp.int32), tei)
    se_ref[...] = tei


def _run_router(xf, gate_W):
    return pl.pallas_call(
        _router_body,
        grid=(1,),
        in_specs=[
            pl.BlockSpec((_T, _H), lambda i: (0, 0)),
            pl.BlockSpec((_E, _H), lambda i: (0, 0)),
            pl.BlockSpec((512, 512), lambda i: (0, 0)),
            pl.BlockSpec((_E, _E), lambda i: (0, 0)),
        ],
        out_specs=[
            pl.BlockSpec((_T, 1), lambda i: (0, 0)),
            pl.BlockSpec((_T, 1), lambda i: (0, 0)),
            pl.BlockSpec((_T, 1), lambda i: (0, 0)),
            pl.BlockSpec((_T, 1), lambda i: (0, 0)),
            pl.BlockSpec((1, 64), lambda i: (0, 0)),
        ],
        out_shape=[
            jax.ShapeDtypeStruct((_T, 1), jnp.int32),
            jax.ShapeDtypeStruct((_T, 1), jnp.int32),
            jax.ShapeDtypeStruct((_T, 1), jnp.float32),
            jax.ShapeDtypeStruct((_T, 1), jnp.float32),
            jax.ShapeDtypeStruct((1, 64), jnp.int32),
        ],
    )(xf, gate_W, jnp.asarray(_TRI, jnp.bfloat16),
      jnp.asarray(_LI, jnp.float32))


def _scatter_body(p1_hbm, p2_hbm, w1_hbm, w2_hbm, tid_hbm, wout_hbm,
                  p1v, p2v, w1v, w2v, tidb, wb):
    cid = lax.axis_index("c")
    sid = lax.axis_index("s")

    @pl.when(jnp.logical_and(cid == 0, sid == 0))
    def _work():
        pltpu.sync_copy(p1_hbm, p1v)
        pltpu.sync_copy(p2_hbm, p2v)
        pltpu.sync_copy(w1_hbm, w1v)
        pltpu.sync_copy(w2_hbm, w2v)

        iota = lax.iota(jnp.int32, 16)
        zi = jnp.zeros((16,), jnp.int32)
        zf = jnp.zeros((16,), jnp.float32)

        def _init(i, _):
            base = i * 16
            tidb[pl.ds(base, 16)] = zi
            wb[pl.ds(base, 16)] = zf
            return 0

        lax.fori_loop(0, _M // 16, _init, 0)

        def _scat(n, _):
            base = n * 16
            tid = base + iota
            idx1 = p1v[pl.ds(base, 16)]
            plsc.store_scatter(tidb, [idx1], tid)
            plsc.store_scatter(wb, [idx1], w1v[pl.ds(base, 16)])
            idx2 = p2v[pl.ds(base, 16)]
            plsc.store_scatter(tidb, [idx2], tid)
            plsc.store_scatter(wb, [idx2], w2v[pl.ds(base, 16)])
            return 0

        lax.fori_loop(0, _T // 16, _scat, 0)
        pltpu.sync_copy(tidb, tid_hbm)
        pltpu.sync_copy(wb, wout_hbm)


def _run_scatter(pos1, pos2, w1, w2):
    mesh = plsc.VectorSubcoreMesh(core_axis_name="c", subcore_axis_name="s")
    f = pl.kernel(
        _scatter_body,
        mesh=mesh,
        compiler_params=_SC_PARAMS,
        out_type=[
            jax.ShapeDtypeStruct((_M,), jnp.int32),
            jax.ShapeDtypeStruct((_M,), jnp.float32),
        ],
        scratch_types=[
            pltpu.VMEM((_T,), jnp.int32),
            pltpu.VMEM((_T,), jnp.int32),
            pltpu.VMEM((_T,), jnp.float32),
            pltpu.VMEM((_T,), jnp.float32),
            pltpu.VMEM((_M,), jnp.int32),
            pltpu.VMEM((_M,), jnp.float32),
        ],
    )
    return f(pos1, pos2, w1, w2)


def _gather_body(xb_hbm, tid_hbm, xg_hbm, idx0, idx1, idx2, idx3,
                 idx4, idx5, rows0, rows1, g0, g1, s0, s1):
    wid = lax.axis_index("s") * 2 + lax.axis_index("c")
    base = wid * _ROWS_W
    idxs = (idx0, idx1, idx2, idx3, idx4, idx5)
    rows = (rows0, rows1)
    gsem = (g0, g1)
    ssem = (s0, s1)
    nch = _ROWS_W // _GCHUNK
    for ch in range(nch):
        pltpu.sync_copy(
            tid_hbm.at[pl.ds(base + ch * _GCHUNK, _GCHUNK)], idxs[ch])
    gcp = [None] * nch
    wcp = [None] * nch
    gcp[0] = pltpu.async_copy(xb_hbm.at[idx0], rows0, g0)
    gcp[1] = pltpu.async_copy(xb_hbm.at[idx1], rows1, g1)
    for ch in range(nch):
        b = ch % 2
        gcp[ch].wait()
        wcp[ch] = pltpu.async_copy(
            rows[b], xg_hbm.at[pl.ds(base + ch * _GCHUNK, _GCHUNK)],
            ssem[b])
        if ch + 2 < nch:
            wcp[ch].wait()
            gcp[ch + 2] = pltpu.async_copy(
                xb_hbm.at[idxs[ch + 2]], rows[b], gsem[b])
    wcp[nch - 2].wait()
    wcp[nch - 1].wait()


def _run_gather(xb, tid_sorted):
    mesh = plsc.VectorSubcoreMesh(core_axis_name="c", subcore_axis_name="s")
    f = pl.kernel(
        _gather_body,
        mesh=mesh,
        compiler_params=_SC_PARAMS,
        out_type=jax.ShapeDtypeStruct((_M, _H), jnp.float32),
        scratch_types=[
            pltpu.VMEM((_GCHUNK,), jnp.int32),
            pltpu.VMEM((_GCHUNK,), jnp.int32),
            pltpu.VMEM((_GCHUNK,), jnp.int32),
            pltpu.VMEM((_GCHUNK,), jnp.int32),
            pltpu.VMEM((_GCHUNK,), jnp.int32),
            pltpu.VMEM((_GCHUNK,), jnp.int32),
            pltpu.VMEM((_GCHUNK, _H), jnp.float32),
            pltpu.VMEM((_GCHUNK, _H), jnp.float32),
            pltpu.SemaphoreType.DMA,
            pltpu.SemaphoreType.DMA,
            pltpu.SemaphoreType.DMA,
            pltpu.SemaphoreType.DMA,
        ],
    )
    return f(xb, tid_sorted)


def _gmm_body(se_ref, xg_ref, w_ref, wg_ref, wu_ref, wd_ref, d_ref):
    m = pl.program_id(0)
    f = pl.program_id(1)
    nr = se_ref[_NT]

    @pl.when(m < nr)
    def _compute():
        x = xg_ref[...].astype(jnp.bfloat16)
        g = lax.dot_general(x, wg_ref[0], (((1,), (1,)), ((), ())),
                            preferred_element_type=jnp.float32)
        u = lax.dot_general(x, wu_ref[0], (((1,), (1,)), ((), ())),
                            preferred_element_type=jnp.float32)
        gu = (jax.nn.silu(g) * u * w_ref[...]).astype(jnp.bfloat16)
        d = lax.dot_general(gu, wd_ref[0], (((1,), (1,)), ((), ())),
                            preferred_element_type=jnp.float32)

        @pl.when(f == 0)
        def _set():
            d_ref[...] = d

        @pl.when(f > 0)
        def _acc():
            d_ref[...] += d


def _run_gmm(se, xg, w_sorted, rWg, rWu, rWd):
    grid = (_NT, _F // _F_BLK)
    spec = pltpu.PrefetchScalarGridSpec(
        num_scalar_prefetch=1,
        grid=grid,
        in_specs=[
            pl.BlockSpec((_TILE, _H), lambda m, f, se: (m, 0)),
            pl.BlockSpec((_TILE, 1), lambda m, f, se: (m, 0)),
            pl.BlockSpec((1, _F_BLK, _H), lambda m, f, se: (se[m], f, 0)),
            pl.BlockSpec((1, _F_BLK, _H), lambda m, f, se: (se[m], f, 0)),
            pl.BlockSpec((1, _H, _F_BLK), lambda m, f, se: (se[m], 0, f)),
        ],
        out_specs=pl.BlockSpec((_TILE, _H), lambda m, f, se: (m, 0)),
    )
    return pl.pallas_call(
        _gmm_body,
        grid_spec=spec,
        out_shape=jax.ShapeDtypeStruct((_M, _H), jnp.float32),
        compiler_params=pltpu.CompilerParams(
            dimension_semantics=("arbitrary", "arbitrary"),
        ),
    )(se, xg, w_sorted, rWg, rWu, rWd)


def _shared_body(x_ref, wg_ref, wu_ref, wd_ref, out_ref):
    e = pl.program_id(0)
    f = pl.program_id(1)
    x = x_ref[...]
    g = lax.dot_general(x, wg_ref[0], (((1,), (1,)), ((), ())),
                        preferred_element_type=jnp.float32)
    u = lax.dot_general(x, wu_ref[0], (((1,), (1,)), ((), ())),
                        preferred_element_type=jnp.float32)
    gu = (jax.nn.silu(g) * u * 0.5).astype(jnp.bfloat16)
    d = lax.dot_general(gu, wd_ref[0], (((1,), (1,)), ((), ())),
                        preferred_element_type=jnp.float32)

    @pl.when(jnp.logical_and(e == 0, f == 0))
    def _set():
        out_ref[...] = d

    @pl.when(jnp.logical_or(e > 0, f > 0))
    def _acc():
        out_ref[...] += d


def _run_shared(xb, sWg, sWu, sWd):
    grid = (2, _F // _F_BLK)
    return pl.pallas_call(
        _shared_body,
        grid=grid,
        in_specs=[
            pl.BlockSpec((_T, _H), lambda e, f: (0, 0)),
            pl.BlockSpec((1, _F_BLK, _H), lambda e, f: (e, f, 0)),
            pl.BlockSpec((1, _F_BLK, _H), lambda e, f: (e, f, 0)),
            pl.BlockSpec((1, _H, _F_BLK), lambda e, f: (e, 0, f)),
        ],
        out_specs=pl.BlockSpec((_T, _H), lambda e, f: (0, 0)),
        out_shape=jax.ShapeDtypeStruct((_T, _H), jnp.float32),
        compiler_params=pltpu.CompilerParams(
            dimension_semantics=("arbitrary", "arbitrary"),
        ),
    )(xb, sWg, sWu, sWd)


def _combine_body(d_hbm, ds_hbm, p1_hbm, p2_hbm, out_hbm, p1v, p2v,
                  idx0, idx1, rows0, rows1, dsv0, dsv1, outv0, outv1,
                  g0, g1, l0, l1, s0, s1):
    wid = lax.axis_index("s") * 2 + lax.axis_index("c")
    tb = wid * _TOK_W
    pltpu.sync_copy(p1_hbm.at[pl.ds(tb, _TOK_W)], p1v)
    pltpu.sync_copy(p2_hbm.at[pl.ds(tb, _TOK_W)], p2v)

    idx = (idx0, idx1)
    rows = (rows0, rows1)
    dsv = (dsv0, dsv1)
    outv = (outv0, outv1)
    gsem = (g0, g1)
    lsem = (l0, l1)
    ssem = (s0, s1)
    nch = _TOK_W // _CCHUNK
    gcp = [None] * nch
    lcp = [None] * nch
    scp = [None] * nch

    def _start(ch):
        t0 = ch * _CCHUNK
        b = ch % 2
        idx[b][pl.ds(0, 16)] = p1v[pl.ds(t0, 16)]
        idx[b][pl.ds(16, 16)] = p2v[pl.ds(t0, 16)]
        gcp[ch] = pltpu.async_copy(d_hbm.at[idx[b]], rows[b], gsem[b])
        lcp[ch] = pltpu.async_copy(
            ds_hbm.at[pl.ds(tb + t0, _CCHUNK)], dsv[b], lsem[b])

    _start(0)
    for ch in range(nch):
        b = ch % 2
        gcp[ch].wait()
        lcp[ch].wait()
        if ch + 1 < nch:
            _start(ch + 1)
        if ch >= 2:
            scp[ch - 2].wait()

        def _tok(t, _):
            def _col(c, _):
                cs = c * 16
                acc = (rows[b][t, pl.ds(cs, 16)]
                       + rows[b][16 + t, pl.ds(cs, 16)]
                       + dsv[b][t, pl.ds(cs, 16)])
                outv[b][t, pl.ds(cs, 16)] = acc
                return 0

            lax.fori_loop(0, _H // 16, _col, 0)
            return 0

        lax.fori_loop(0, _CCHUNK, _tok, 0)
        scp[ch] = pltpu.async_copy(
            outv[b], out_hbm.at[pl.ds(tb + ch * _CCHUNK, _CCHUNK)],
            ssem[b])
    scp[nch - 2].wait()
    scp[nch - 1].wait()


def _run_combine(d, ds, pos1, pos2):
    mesh = plsc.VectorSubcoreMesh(core_axis_name="c", subcore_axis_name="s")
    f = pl.kernel(
        _combine_body,
        mesh=mesh,
        compiler_params=_SC_PARAMS,
        out_type=jax.ShapeDtypeStruct((_T, _H), jnp.float32),
        scratch_types=[
            pltpu.VMEM((_TOK_W,), jnp.int32),
            pltpu.VMEM((_TOK_W,), jnp.int32),
            pltpu.VMEM((2 * _CCHUNK,), jnp.int32),
            pltpu.VMEM((2 * _CCHUNK,), jnp.int32),
            pltpu.VMEM((2 * _CCHUNK, _H), jnp.float32),
            pltpu.VMEM((2 * _CCHUNK, _H), jnp.float32),
            pltpu.VMEM((_CCHUNK, _H), jnp.float32),
            pltpu.VMEM((_CCHUNK, _H), jnp.float32),
            pltpu.VMEM((_CCHUNK, _H), jnp.float32),
            pltpu.VMEM((_CCHUNK, _H), jnp.float32),
            pltpu.SemaphoreType.DMA,
            pltpu.SemaphoreType.DMA,
            pltpu.SemaphoreType.DMA,
            pltpu.SemaphoreType.DMA,
            pltpu.SemaphoreType.DMA,
            pltpu.SemaphoreType.DMA,
        ],
    )
    return f(d, ds, pos1, pos2)


@functools.partial(jax.jit, static_argnames=())
def kernel(hidden_states, gate_W, sWg, sWu, sWd, rWg, rWu, rWd):
    b, s, h = hidden_states.shape
    xf = hidden_states.reshape(s, h)
    xb = xf.astype(jnp.bfloat16)

    pos1, pos2, w1, w2, se = _run_router(xf, gate_W)
    pos1 = pos1.reshape(_T)
    pos2 = pos2.reshape(_T)
    tid_sorted, w_sorted = _run_scatter(pos1, pos2, w1.reshape(_T),
                                        w2.reshape(_T))
    xg = _run_gather(xf, tid_sorted)
    ds = _run_shared(xb, sWg.astype(jnp.bfloat16),
                     sWu.astype(jnp.bfloat16), sWd.astype(jnp.bfloat16))
    d = _run_gmm(se.reshape(64), xg, w_sorted.reshape(_M, 1),
                 rWg.astype(jnp.bfloat16), rWu.astype(jnp.bfloat16),
                 rWd.astype(jnp.bfloat16))
    out = _run_combine(d, ds, pos1, pos2)
    return out.reshape(b, s, h)


# speedup vs baseline: 1.4990x; 1.4990x over previous
import functools

import jax
import jax.numpy as jnp
from jax.experimental import pallas as pl
from jax.experimental.pallas import tpu as pltpu

_HIDDEN = 768
_INTER = 1536
_N_SHARED = 2
_N_ROUTED = 8
_TOP_K = 2
_F_BLK = 768


def _moe_body(x_ref, gw_ref, wg_ref, wu_ref, wd_ref, out_ref, w_scr):
    e = pl.program_id(0)
    f = pl.program_id(1)

    x = x_ref[...]

    @pl.when(jnp.logical_and(e == 0, f == 0))
    def _router():
        logits = jax.lax.dot_general(
            x, gw_ref[...], (((1,), (1,)), ((), ())),
            preferred_element_type=jnp.float32)
        p = jax.nn.softmax(logits, axis=-1)
        i1 = jnp.argmax(p, axis=-1)
        lane = jax.lax.broadcasted_iota(jnp.int32, p.shape, 1)
        m1h = (lane == i1[:, None])
        m1 = jnp.max(p, axis=-1, keepdims=True)
        p2 = jnp.where(m1h, -jnp.inf, p)
        i2 = jnp.argmax(p2, axis=-1)
        m2h = (lane == i2[:, None])
        m2 = jnp.max(p2, axis=-1, keepdims=True)
        denom = m1 + m2 + 1e-8
        w_scr[...] = jnp.where(m1h | m2h, p / denom, 0.0)
        out_ref[...] = jnp.zeros_like(out_ref)

    onehot = (jax.lax.broadcasted_iota(jnp.int32, (2048, _N_ROUTED), 1)
              == e)
    w_col = jnp.sum(jnp.where(onehot, w_scr[...], 0.0), axis=1,
                    keepdims=True)
    w_col = jnp.where(e < _N_ROUTED, w_col, 1.0 / _N_SHARED)

    g = jax.lax.dot_general(x, wg_ref[0], (((1,), (1,)), ((), ())),
                            preferred_element_type=jnp.float32)
    u = jax.lax.dot_general(x, wu_ref[0], (((1,), (1,)), ((), ())),
                            preferred_element_type=jnp.float32)
    gu = (jax.nn.silu(g) * u * w_col).astype(jnp.bfloat16)
    d = jax.lax.dot_general(gu, wd_ref[0], (((1,), (1,)), ((), ())),
                            preferred_element_type=jnp.float32)
    out_ref[...] += d


@functools.partial(jax.jit, static_argnames=())
def kernel(hidden_states, gate_W, sWg, sWu, sWd, rWg, rWu, rWd):
    b, s, h = hidden_states.shape
    xf = hidden_states.reshape(s, h).astype(jnp.bfloat16)
    Wg = jnp.concatenate([rWg, sWg], axis=0).astype(jnp.bfloat16)
    Wu = jnp.concatenate([rWu, sWu], axis=0).astype(jnp.bfloat16)
    Wd = jnp.concatenate([rWd, sWd], axis=0).astype(jnp.bfloat16)

    n_e = _N_ROUTED + _N_SHARED
    grid = (n_e, _INTER // _F_BLK)

    out = pl.pallas_call(
        _moe_body,
        grid=grid,
        in_specs=[
            pl.BlockSpec((s, h), lambda e, f: (0, 0)),
            pl.BlockSpec((_N_ROUTED, h), lambda e, f: (0, 0)),
            pl.BlockSpec((1, _F_BLK, h), lambda e, f: (e, f, 0)),
            pl.BlockSpec((1, _F_BLK, h), lambda e, f: (e, f, 0)),
            pl.BlockSpec((1, h, _F_BLK), lambda e, f: (e, 0, f)),
        ],
        out_specs=pl.BlockSpec((s, h), lambda e, f: (0, 0)),
        out_shape=jax.ShapeDtypeStruct((s, h), jnp.float32),
        scratch_shapes=[pltpu.VMEM((s, _N_ROUTED), jnp.float32)],
        compiler_params=pltpu.CompilerParams(
            dimension_semantics=("arbitrary", "arbitrary"),
        ),
    )(xf, gate_W.astype(jnp.bfloat16), Wg, Wu, Wd)
    return out.reshape(b, s, h)


# speedup vs baseline: 1.5441x; 1.0301x over previous
import functools

import jax
import jax.numpy as jnp
from jax.experimental import pallas as pl
from jax.experimental.pallas import tpu as pltpu

_HIDDEN = 768
_INTER = 1536
_N_SHARED = 2
_N_ROUTED = 8
_TOP_K = 2
_F_BLK = 1536


def _moe_body(x_ref, gw_ref, wg_ref, wu_ref, wd_ref, out_ref, w_scr):
    e = pl.program_id(0)
    f = pl.program_id(1)

    x = x_ref[...]

    @pl.when(jnp.logical_and(e == 0, f == 0))
    def _router():
        logits = jax.lax.dot_general(
            x, gw_ref[...], (((1,), (1,)), ((), ())),
            preferred_element_type=jnp.float32)
        p = jax.nn.softmax(logits, axis=-1)
        i1 = jnp.argmax(p, axis=-1)
        lane = jax.lax.broadcasted_iota(jnp.int32, p.shape, 1)
        m1h = (lane == i1[:, None])
        m1 = jnp.max(p, axis=-1, keepdims=True)
        p2 = jnp.where(m1h, -jnp.inf, p)
        i2 = jnp.argmax(p2, axis=-1)
        m2h = (lane == i2[:, None])
        m2 = jnp.max(p2, axis=-1, keepdims=True)
        denom = m1 + m2 + 1e-8
        w_scr[...] = jnp.where(m1h | m2h, p / denom, 0.0)
        out_ref[...] = jnp.zeros_like(out_ref)

    onehot = (jax.lax.broadcasted_iota(jnp.int32, (2048, _N_ROUTED), 1)
              == e)
    w_col = jnp.sum(jnp.where(onehot, w_scr[...], 0.0), axis=1,
                    keepdims=True)
    w_col = jnp.where(e < _N_ROUTED, w_col, 1.0 / _N_SHARED)

    g = jax.lax.dot_general(x, wg_ref[0], (((1,), (1,)), ((), ())),
                            preferred_element_type=jnp.float32)
    u = jax.lax.dot_general(x, wu_ref[0], (((1,), (1,)), ((), ())),
                            preferred_element_type=jnp.float32)
    gu = (jax.nn.silu(g) * u * w_col).astype(jnp.bfloat16)
    d = jax.lax.dot_general(gu, wd_ref[0], (((1,), (1,)), ((), ())),
                            preferred_element_type=jnp.float32)
    out_ref[...] += d


@functools.partial(jax.jit, static_argnames=())
def kernel(hidden_states, gate_W, sWg, sWu, sWd, rWg, rWu, rWd):
    b, s, h = hidden_states.shape
    xf = hidden_states.reshape(s, h).astype(jnp.bfloat16)
    Wg = jnp.concatenate([rWg, sWg], axis=0).astype(jnp.bfloat16)
    Wu = jnp.concatenate([rWu, sWu], axis=0).astype(jnp.bfloat16)
    Wd = jnp.concatenate([rWd, sWd], axis=0).astype(jnp.bfloat16)

    n_e = _N_ROUTED + _N_SHARED
    grid = (n_e, _INTER // _F_BLK)

    out = pl.pallas_call(
        _moe_body,
        grid=grid,
        in_specs=[
            pl.BlockSpec((s, h), lambda e, f: (0, 0)),
            pl.BlockSpec((_N_ROUTED, h), lambda e, f: (0, 0)),
            pl.BlockSpec((1, _F_BLK, h), lambda e, f: (e, f, 0)),
            pl.BlockSpec((1, _F_BLK, h), lambda e, f: (e, f, 0)),
            pl.BlockSpec((1, h, _F_BLK), lambda e, f: (e, 0, f)),
        ],
        out_specs=pl.BlockSpec((s, h), lambda e, f: (0, 0)),
        out_shape=jax.ShapeDtypeStruct((s, h), jnp.float32),
        scratch_shapes=[pltpu.VMEM((s, _N_ROUTED), jnp.float32)],
        compiler_params=pltpu.CompilerParams(
            dimension_semantics=("arbitrary", "arbitrary"),
        ),
    )(xf, gate_W.astype(jnp.bfloat16), Wg, Wu, Wd)
    return out.reshape(b, s, h)
